# baseline (speedup 1.0000x reference)
import jax
import jax.numpy as jnp
from jax.experimental import pallas as pl
from jax.experimental.pallas import tpu as pltpu

_C = 10
_F = 128
_B = 2048


def _body(x_ref, lab_ref, cen_ref, out_ref, s_ref, n_ref):
    i = pl.program_id(0)

    @pl.when(i == 0)
    def _():
        s_ref[...] = jnp.zeros_like(s_ref)
        n_ref[...] = jnp.zeros_like(n_ref)

    x = x_ref[...]
    labels = lab_ref[...]
    cen = cen_ref[...]
    contract = (((1,), (1,)), ((), ()))
    hi = jax.lax.Precision.HIGHEST
    dots = jax.lax.dot_general(x, cen, contract, precision=hi,
                               preferred_element_type=jnp.float32)
    xx = jax.lax.dot_general(x * x, jnp.ones((1, _F), jnp.float32),
                             contract, precision=hi,
                             preferred_element_type=jnp.float32)
    cn = jax.lax.dot_general(jnp.ones((1, _F), jnp.float32), cen * cen,
                             contract, precision=hi,
                             preferred_element_type=jnp.float32)
    d2 = xx - 2.0 * dots + cn
    dist = jnp.sqrt(jnp.maximum(d2, 0.0))
    onehot = (labels == jax.lax.broadcasted_iota(jnp.int32, (1, _C), 1)
              ).astype(jnp.float32)
    s_ref[...] += jnp.sum(dist * onehot, axis=0, keepdims=True)
    n_ref[...] += jnp.sum(onehot, axis=0, keepdims=True)

    @pl.when(i == pl.num_programs(0) - 1)
    def _():
        s = s_ref[...]
        n = n_ref[...]
        out_ref[...] = jnp.sum(jnp.where(n > 0, s / n, 0.0),
                               axis=1, keepdims=True)


def kernel(x, labels, centers):
    batch = x.shape[0]
    labels2 = labels.astype(jnp.int32).reshape(batch, 1)
    out = pl.pallas_call(
        _body,
        grid=(batch // _B,),
        in_specs=[
            pl.BlockSpec((_B, _F), lambda i: (i, 0)),
            pl.BlockSpec((_B, 1), lambda i: (i, 0)),
            pl.BlockSpec((_C, _F), lambda i: (0, 0)),
        ],
        out_specs=pl.BlockSpec((1, 1), lambda i: (0, 0)),
        out_shape=jax.ShapeDtypeStruct((1, 1), jnp.float32),
        scratch_shapes=[
            pltpu.VMEM((1, _C), jnp.float32),
            pltpu.VMEM((1, _C), jnp.float32),
        ],
        compiler_params=pltpu.CompilerParams(
            dimension_semantics=("arbitrary",)),
    )(x, labels2, centers)
    return out[0, 0]


# speedup vs baseline: 2.5584x; 2.5584x over previous
import jax
import jax.numpy as jnp
from jax.experimental import pallas as pl
from jax.experimental.pallas import tpu as pltpu

_C = 10
_F = 128
_B = 2048


def _body(x_ref, lab_ref, cen_ref, out_ref, s_ref, n_ref):
    i = pl.program_id(0)

    @pl.when(i == 0)
    def _():
        s_ref[...] = jnp.zeros_like(s_ref)
        n_ref[...] = jnp.zeros_like(n_ref)

    x = x_ref[...]
    lab = lab_ref[0]
    cen = cen_ref[...]
    contract = (((1,), (1,)), ((), ()))
    dots = jax.lax.dot_general(cen, x, contract,
                               preferred_element_type=jnp.float32)
    xx = jax.lax.dot_general(jnp.ones((1, _F), jnp.float32), x * x,
                             contract,
                             preferred_element_type=jnp.float32)
    cn = jax.lax.dot_general(cen * cen, jnp.ones((1, _F), jnp.float32),
                             contract,
                             preferred_element_type=jnp.float32)
    d2 = xx - 2.0 * dots + cn
    dist = jnp.sqrt(jnp.maximum(d2, 0.0))
    onehot = (lab == jax.lax.broadcasted_iota(jnp.int32, (_C, _B), 0)
              ).astype(jnp.float32)
    s_ref[...] += jnp.sum(dist * onehot, axis=1, keepdims=True)
    n_ref[...] += jnp.sum(onehot, axis=1, keepdims=True)

    @pl.when(i == pl.num_programs(0) - 1)
    def _():
        s = s_ref[...]
        n = n_ref[...]
        out_ref[...] = jnp.sum(jnp.where(n > 0, s / n, 0.0),
                               axis=0, keepdims=True)


def kernel(x, labels, centers):
    batch = x.shape[0]
    grid = batch // _B
    labels3 = labels.astype(jnp.int32).reshape(grid, 1, _B)
    out = pl.pallas_call(
        _body,
        grid=(grid,),
        in_specs=[
            pl.BlockSpec((_B, _F), lambda i: (i, 0)),
            pl.BlockSpec((1, 1, _B), lambda i: (i, 0, 0)),
            pl.BlockSpec((_C, _F), lambda i: (0, 0)),
        ],
        out_specs=pl.BlockSpec((1, 1), lambda i: (0, 0)),
        out_shape=jax.ShapeDtypeStruct((1, 1), jnp.float32),
        scratch_shapes=[
            pltpu.VMEM((_C, 1), jnp.float32),
            pltpu.VMEM((_C, 1), jnp.float32),
        ],
        compiler_params=pltpu.CompilerParams(
            dimension_semantics=("arbitrary",)),
    )(x, labels3, centers)
    return out[0, 0]


# speedup vs baseline: 3.4061x; 1.3313x over previous
import jax
import jax.numpy as jnp
from jax.experimental import pallas as pl
from jax.experimental.pallas import tpu as pltpu

_C = 10
_F = 128
_B = 4096


def _body(x_ref, lab_ref, cen_ref, out_ref, s_ref, n_ref):
    i = pl.program_id(0)

    @pl.when(i == 0)
    def _():
        s_ref[...] = jnp.zeros_like(s_ref)
        n_ref[...] = jnp.zeros_like(n_ref)

    x = x_ref[...]
    lab = lab_ref[0]
    cen = cen_ref[...]
    contract = (((1,), (1,)), ((), ()))
    dots = jax.lax.dot_general(cen, x, contract,
                               preferred_element_type=jnp.float32)
    xx = jax.lax.dot_general(jnp.ones((1, _F), jnp.float32), x * x,
                             contract,
                             preferred_element_type=jnp.float32)
    cn = jax.lax.dot_general(cen * cen, jnp.ones((1, _F), jnp.float32),
                             contract,
                             preferred_element_type=jnp.float32)
    d2 = xx - 2.0 * dots + cn
    dist = jnp.sqrt(jnp.maximum(d2, 0.0))
    onehot = (lab == jax.lax.broadcasted_iota(jnp.int32, (_C, _B), 0)
              ).astype(jnp.float32)
    s_ref[...] += jnp.sum(dist * onehot, axis=1, keepdims=True)
    n_ref[...] += jnp.sum(onehot, axis=1, keepdims=True)

    @pl.when(i == pl.num_programs(0) - 1)
    def _():
        s = s_ref[...]
        n = n_ref[...]
        out_ref[...] = jnp.sum(jnp.where(n > 0, s / n, 0.0),
                               axis=0, keepdims=True)


def kernel(x, labels, centers):
    batch = x.shape[0]
    grid = batch // _B
    labels3 = labels.astype(jnp.int32).reshape(grid, 1, _B)
    out = pl.pallas_call(
        _body,
        grid=(grid,),
        in_specs=[
            pl.BlockSpec((_B, _F), lambda i: (i, 0)),
            pl.BlockSpec((1, 1, _B), lambda i: (i, 0, 0)),
            pl.BlockSpec((_C, _F), lambda i: (0, 0)),
        ],
        out_specs=pl.BlockSpec((1, 1), lambda i: (0, 0)),
        out_shape=jax.ShapeDtypeStruct((1, 1), jnp.float32),
        scratch_shapes=[
            pltpu.VMEM((_C, 1), jnp.float32),
            pltpu.VMEM((_C, 1), jnp.float32),
        ],
        compiler_params=pltpu.CompilerParams(
            dimension_semantics=("arbitrary",)),
    )(x, labels3, centers)
    return out[0, 0]


# speedup vs baseline: 3.7915x; 1.1132x over previous
import jax
import jax.numpy as jnp
from jax.experimental import pallas as pl
from jax.experimental.pallas import tpu as pltpu

_C = 10
_F = 128
_B = 8192


def _body(x_ref, lab_ref, cen_ref, out_ref, s_ref, n_ref):
    i = pl.program_id(0)

    @pl.when(i == 0)
    def _():
        s_ref[...] = jnp.zeros_like(s_ref)
        n_ref[...] = jnp.zeros_like(n_ref)

    x = x_ref[...]
    lab = lab_ref[0]
    cen = cen_ref[...]
    contract = (((1,), (1,)), ((), ()))
    dots = jax.lax.dot_general(cen, x, contract,
                               preferred_element_type=jnp.float32)
    xx = jax.lax.dot_general(jnp.ones((1, _F), jnp.float32), x * x,
                             contract,
                             preferred_element_type=jnp.float32)
    cn = jax.lax.dot_general(cen * cen, jnp.ones((1, _F), jnp.float32),
                             contract,
                             preferred_element_type=jnp.float32)
    d2 = xx - 2.0 * dots + cn
    dist = jnp.sqrt(jnp.maximum(d2, 0.0))
    onehot = (lab == jax.lax.broadcasted_iota(jnp.int32, (_C, _B), 0)
              ).astype(jnp.float32)
    s_ref[...] += jnp.sum(dist * onehot, axis=1, keepdims=True)
    n_ref[...] += jnp.sum(onehot, axis=1, keepdims=True)

    @pl.when(i == pl.num_programs(0) - 1)
    def _():
        s = s_ref[...]
        n = n_ref[...]
        out_ref[...] = jnp.sum(jnp.where(n > 0, s / n, 0.0),
                               axis=0, keepdims=True)


def kernel(x, labels, centers):
    batch = x.shape[0]
    grid = batch // _B
    labels3 = labels.astype(jnp.int32).reshape(grid, 1, _B)
    out = pl.pallas_call(
        _body,
        grid=(grid,),
        in_specs=[
            pl.BlockSpec((_B, _F), lambda i: (i, 0)),
            pl.BlockSpec((1, 1, _B), lambda i: (i, 0, 0)),
            pl.BlockSpec((_C, _F), lambda i: (0, 0)),
        ],
        out_specs=pl.BlockSpec((1, 1), lambda i: (0, 0)),
        out_shape=jax.ShapeDtypeStruct((1, 1), jnp.float32),
        scratch_shapes=[
            pltpu.VMEM((_C, 1), jnp.float32),
            pltpu.VMEM((_C, 1), jnp.float32),
        ],
        compiler_params=pltpu.CompilerParams(
            dimension_semantics=("arbitrary",)),
    )(x, labels3, centers)
    return out[0, 0]
